# Initial kernel scaffold; baseline (speedup 1.0000x reference)
import jax
import jax.numpy as jnp
from jax.experimental import pallas as pl


def kernel(x1, edge_index1, x2, edge_index2, group_index, W1, a1_src, a1_dst, b1, W2, a2_src, a2_dst, b2, Wfc, bfc):
    raise NotImplementedError("write your pallas kernel here")


# speedup vs baseline: 16.3911x; 16.3911x over previous
import functools

import jax
import jax.numpy as jnp
from jax import lax
from jax.experimental import pallas as pl
from jax.experimental.pallas import tpu as pltpu
from jax.experimental.pallas import tpu_sc as plsc

N = 10000
D = 128
G = 100
P = 100
LANES = 16
NTILES = 16
NCORES = 2
NS = 10240
TRASH = N
B = 128
ROWS_PER_TILE = NS // NTILES
ZCHUNKS = ROWS_PER_TILE // LANES
WCHUNKS = ROWS_PER_TILE // B


def _proj_body(x_ref, w_ref, asrc_ref, adst_ref, h_ref, as_ref, ad_ref):
    h = jnp.dot(x_ref[...], w_ref[...], preferred_element_type=jnp.float32)
    h_ref[...] = h
    as_ref[...] = jnp.sum(h * asrc_ref[...], axis=1, keepdims=True)
    ad_ref[...] = jnp.sum(h * adst_ref[...], axis=1, keepdims=True)


def _project(x, w, a_src, a_dst):
    n = x.shape[0]
    blk = 2000
    grid = n // blk
    return pl.pallas_call(
        _proj_body,
        grid=(grid,),
        in_specs=[
            pl.BlockSpec((blk, D), lambda i: (i, 0)),
            pl.BlockSpec((D, D), lambda i: (0, 0)),
            pl.BlockSpec((1, D), lambda i: (0, 0)),
            pl.BlockSpec((1, D), lambda i: (0, 0)),
        ],
        out_specs=[
            pl.BlockSpec((blk, D), lambda i: (i, 0)),
            pl.BlockSpec((blk, 1), lambda i: (i, 0)),
            pl.BlockSpec((blk, 1), lambda i: (i, 0)),
        ],
        out_shape=[
            jax.ShapeDtypeStruct((n, D), jnp.float32),
            jax.ShapeDtypeStruct((n, 1), jnp.float32),
            jax.ShapeDtypeStruct((n, 1), jnp.float32),
        ],
    )(x, w, a_src, a_dst)


def _group_body(x2_ref, w2_ref, a2s_ref, a2d_ref, b2_ref, ei2_ref,
                xg_ref, am_ref):
    h2 = jnp.dot(x2_ref[...], w2_ref[...], preferred_element_type=jnp.float32)
    as2 = jnp.sum(h2 * a2s_ref[...], axis=1)
    ad2 = jnp.sum(h2 * a2d_ref[...], axis=1)
    ei = ei2_ref[...]
    eg = ei.shape[1]
    gi = lax.broadcasted_iota(jnp.int32, (eg, G), 1)
    src_oh = (ei[0][:, None] == gi).astype(jnp.float32)
    dst_oh = (ei[1][:, None] == gi).astype(jnp.float32)
    m_cnt = lax.dot_general(src_oh, dst_oh, (((0,), (0,)), ((), ())),
                            preferred_element_type=jnp.float32)
    eye = (lax.broadcasted_iota(jnp.int32, (G, G), 0)
           == lax.broadcasted_iota(jnp.int32, (G, G), 1)).astype(jnp.float32)
    mp = m_cnt + eye
    emat = as2[:, None] + ad2[None, :]
    el = jnp.maximum(emat, 0.2 * emat)
    mx = jnp.max(jnp.where(mp > 0, el, -1e30), axis=0)
    z = mp * jnp.exp(el - mx[None, :])
    ssum = jnp.sum(z, axis=0)
    num = lax.dot_general(z, h2, (((0,), (0,)), ((), ())),
                          preferred_element_type=jnp.float32)
    xg = num / (ssum[:, None] + 1e-16) + b2_ref[...][None, :]
    xg_ref[...] = xg
    cnt = jnp.sum(m_cnt, axis=1)
    adj_sum = jnp.dot(m_cnt, xg, preferred_element_type=jnp.float32)
    am_ref[...] = adj_sum / jnp.maximum(cnt, 1.0)[:, None]


def _group_gat(x2, w2, a2_src, a2_dst, b2, ei2):
    return pl.pallas_call(
        _group_body,
        out_shape=[
            jax.ShapeDtypeStruct((G, D), jnp.float32),
            jax.ShapeDtypeStruct((G, D), jnp.float32),
        ],
    )(x2, w2, a2_src, a2_dst, b2, ei2)


def _make_sc_pass1(epad, chunks):
    per_tile = chunks * B
    mesh = plsc.VectorSubcoreMesh(core_axis_name="c", subcore_axis_name="s")

    @functools.partial(
        pl.kernel,
        out_type=jax.ShapeDtypeStruct((NCORES, NS), jnp.float32),
        mesh=mesh,
        compiler_params=pltpu.CompilerParams(needs_layout_passes=False),
        scratch_types=[
            pltpu.VMEM((NS,), jnp.float32),
            pltpu.VMEM((NS,), jnp.float32),
            pltpu.VMEM((B,), jnp.int32),
            pltpu.VMEM((B,), jnp.int32),
            pltpu.VMEM((B,), jnp.float32),
            pltpu.VMEM((ROWS_PER_TILE,), jnp.float32),
            pltpu.VMEM_SHARED((NS,), jnp.float32),
        ],
    )
    def pass1(src_hbm, dst_hbm, as_hbm, ad_hbm, s_out,
              as_loc, ad_loc, src_b, dst_b, ex_b, zbuf, s_sh):
        c = lax.axis_index("c")
        t = lax.axis_index("s")
        wid = c * NTILES + t

        def zi(i, _):
            zbuf[pl.ds(i * LANES, LANES)] = jnp.zeros((LANES,), jnp.float32)
            return 0
        lax.fori_loop(0, ZCHUNKS, zi, 0)
        pltpu.sync_copy(zbuf, s_sh.at[pl.ds(t * ROWS_PER_TILE, ROWS_PER_TILE)])

        pltpu.sync_copy(as_hbm, as_loc.at[pl.ds(0, N)])
        pltpu.sync_copy(ad_hbm, ad_loc.at[pl.ds(0, N)])
        for i in range((NS - N) // LANES):
            as_loc[pl.ds(N + i * LANES, LANES)] = jnp.zeros((LANES,), jnp.float32)
            ad_loc[pl.ds(N + i * LANES, LANES)] = jnp.zeros((LANES,), jnp.float32)
        plsc.subcore_barrier()

        base = wid * per_tile

        def chunk(k, _):
            off = base + k * B
            pltpu.sync_copy(src_hbm.at[pl.ds(off, B)], src_b)
            pltpu.sync_copy(dst_hbm.at[pl.ds(off, B)], dst_b)
            for g in range(B // LANES):
                sv = src_b[pl.ds(g * LANES, LANES)]
                dv = dst_b[pl.ds(g * LANES, LANES)]
                e = (plsc.load_gather(as_loc, [sv])
                     + plsc.load_gather(ad_loc, [dv]))
                e = jnp.maximum(e, 0.2 * e)
                ex_b[pl.ds(g * LANES, LANES)] = jnp.exp(e)
            pltpu.sync_copy(ex_b, s_sh.at[dst_b], add=True)
            return 0
        lax.fori_loop(0, chunks, chunk, 0)
        plsc.subcore_barrier()
        pltpu.sync_copy(
            s_sh.at[pl.ds(t * ROWS_PER_TILE, ROWS_PER_TILE)],
            s_out.at[c, pl.ds(t * ROWS_PER_TILE, ROWS_PER_TILE)])

    return pass1


def _make_sc_pass2(epad, chunks):
    per_tile = chunks * B
    mesh = plsc.VectorSubcoreMesh(core_axis_name="c", subcore_axis_name="s")

    @functools.partial(
        pl.kernel,
        out_type=jax.ShapeDtypeStruct((NCORES, NS, D), jnp.float32),
        mesh=mesh,
        compiler_params=pltpu.CompilerParams(needs_layout_passes=False),
        scratch_types=[
            pltpu.VMEM((NS,), jnp.float32),
            pltpu.VMEM((NS,), jnp.float32),
            pltpu.VMEM((NS,), jnp.float32),
            pltpu.VMEM((B,), jnp.float32),
            pltpu.VMEM((B,), jnp.int32),
            pltpu.VMEM((B,), jnp.int32),
            pltpu.VMEM((B,), jnp.float32),
            pltpu.VMEM((B, D), jnp.float32),
            pltpu.SemaphoreType.DMA,
            pltpu.VMEM_SHARED((NS, D), jnp.float32),
        ],
    )
    def pass2(src_hbm, dst_hbm, h_hbm, as_hbm, ad_hbm, s2_hbm, out_hbm,
              as_loc, ad_loc, s_loc, s_tmp, src_b, dst_b, w_b, rows, sem,
              out_sh):
        c = lax.axis_index("c")
        t = lax.axis_index("s")
        wid = c * NTILES + t

        def zr(i, _):
            for cc in range(D // LANES):
                rows[i, pl.ds(cc * LANES, LANES)] = jnp.zeros((LANES,), jnp.float32)
            return 0
        lax.fori_loop(0, B, zr, 0)
        for j in range(WCHUNKS):
            pltpu.sync_copy(rows, out_sh.at[pl.ds(t * ROWS_PER_TILE + j * B, B)])

        pltpu.sync_copy(as_hbm, as_loc.at[pl.ds(0, N)])
        pltpu.sync_copy(ad_hbm, ad_loc.at[pl.ds(0, N)])
        for i in range((NS - N) // LANES):
            as_loc[pl.ds(N + i * LANES, LANES)] = jnp.zeros((LANES,), jnp.float32)
            ad_loc[pl.ds(N + i * LANES, LANES)] = jnp.zeros((LANES,), jnp.float32)
        pltpu.sync_copy(s2_hbm.at[0], s_loc)

        def si(j, _):
            pltpu.sync_copy(s2_hbm.at[1, pl.ds(j * B, B)], s_tmp)
            for g in range(B // LANES):
                sl = pl.ds(j * B + g * LANES, LANES)
                s_loc[sl] = s_loc[sl] + s_tmp[pl.ds(g * LANES, LANES)]
            return 0
        lax.fori_loop(0, NS // B, si, 0)
        plsc.subcore_barrier()

        base = wid * per_tile

        def chunk(k, _):
            off = base + k * B
            pltpu.sync_copy(src_hbm.at[pl.ds(off, B)], src_b)
            pltpu.sync_copy(dst_hbm.at[pl.ds(off, B)], dst_b)
            pltpu.async_copy(h_hbm.at[src_b], rows, sem).wait()
            for g in range(B // LANES):
                sv = src_b[pl.ds(g * LANES, LANES)]
                dv = dst_b[pl.ds(g * LANES, LANES)]
                e = (plsc.load_gather(as_loc, [sv])
                     + plsc.load_gather(ad_loc, [dv]))
                e = jnp.maximum(e, 0.2 * e)
                ex = jnp.exp(e)
                sg = plsc.load_gather(s_loc, [dv])
                w_b[pl.ds(g * LANES, LANES)] = ex / sg

            def scale(i, _):
                bvec = jnp.full((LANES,), i, jnp.int32)
                w = plsc.load_gather(w_b, [bvec])
                for cc in range(D // LANES):
                    sl = pl.ds(cc * LANES, LANES)
                    rows[i, sl] = rows[i, sl] * w
                return 0
            lax.fori_loop(0, B, scale, 0)
            pltpu.sync_copy(rows, out_sh.at[dst_b], add=True)
            return 0
        lax.fori_loop(0, chunks, chunk, 0)
        plsc.subcore_barrier()
        for j in range(WCHUNKS):
            off = t * ROWS_PER_TILE + j * B
            pltpu.sync_copy(out_sh.at[pl.ds(off, B)],
                            out_hbm.at[c, pl.ds(off, B)])

    return pass2


def _final_body(p_ref, b1_ref, xg_ref, am_ref, wfc_ref, bfc_ref, out_ref):
    p = p_ref[...]
    nf = p[0, 0] + p[1, 0] + b1_ref[...][None, :]
    xg = xg_ref[...][0]
    am = am_ref[...][0]
    impg = jnp.sum(nf * xg, axis=1, keepdims=True)
    impa = jnp.sum(nf * am, axis=1, keepdims=True)
    upd = nf + impg * xg + impa * am
    out = jnp.dot(upd, wfc_ref[...], preferred_element_type=jnp.float32)
    out_ref[...] = (out + bfc_ref[...][None, :])[None]


def _final(parts, b1, xg, am, wfc, bfc):
    return pl.pallas_call(
        _final_body,
        grid=(G,),
        in_specs=[
            pl.BlockSpec((2, 1, P, D), lambda g: (0, g, 0, 0)),
            pl.BlockSpec((D,), lambda g: (0,)),
            pl.BlockSpec((1, 1, D), lambda g: (g, 0, 0)),
            pl.BlockSpec((1, 1, D), lambda g: (g, 0, 0)),
            pl.BlockSpec((D, D), lambda g: (0, 0)),
            pl.BlockSpec((D,), lambda g: (0,)),
        ],
        out_specs=pl.BlockSpec((1, P, D), lambda g: (g, 0, 0)),
        out_shape=jax.ShapeDtypeStruct((G, P, D), jnp.float32),
    )(parts, b1, xg.reshape(G, 1, D), am.reshape(G, 1, D), wfc, bfc)


def kernel(x1, edge_index1, x2, edge_index2, group_index,
           W1, a1_src, a1_dst, b1, W2, a2_src, a2_dst, b2, Wfc, bfc):
    e = edge_index1.shape[1]
    e_total = e + N
    chunks = -(-e_total // (NCORES * NTILES * B))
    epad = NCORES * NTILES * B * chunks
    npad = epad - e_total

    loop = jnp.arange(N, dtype=jnp.int32)
    src_all = jnp.concatenate([
        edge_index1[0].astype(jnp.int32), loop,
        jnp.zeros((npad,), jnp.int32)])
    dst_all = jnp.concatenate([
        edge_index1[1].astype(jnp.int32), loop,
        jnp.full((npad,), TRASH, jnp.int32)])

    h, as1, ad1 = _project(x1, W1, a1_src, a1_dst)
    as1 = as1.reshape(N)
    ad1 = ad1.reshape(N)
    s2 = _make_sc_pass1(epad, chunks)(src_all, dst_all, as1, ad1)
    parts = _make_sc_pass2(epad, chunks)(src_all, dst_all, h, as1, ad1, s2)
    xg, am = _group_gat(x2, W2, a2_src, a2_dst, b2, edge_index2)

    parts4 = parts[:, :N, :].reshape(NCORES, G, P, D)
    out1 = _final(parts4, b1, xg, am, Wfc, bfc).reshape(N, D)
    return out1, xg
